# baseline (speedup 1.0000x reference)
import functools

import jax
import jax.numpy as jnp
from jax import lax
from jax.experimental import pallas as pl
from jax.experimental.pallas import tpu as pltpu
from jax.experimental.pallas import tpu_sc as plsc

N = 10000
E = 320000
IN = 128
H = 256
G = 64

NPAD = 10240
HH = H // 2
ETOT = E + N
C = 128
EROWS = 2816
EPAD = EROWS * C
RA = EROWS // 32
RB = EROWS // 16
GR = 8
NBLK = NPAD // 256
NPT = NPAD // 16

_SC_PARAMS = pltpu.CompilerParams(needs_layout_passes=False)


def _sc_mesh():
    return plsc.VectorSubcoreMesh(core_axis_name="c", subcore_axis_name="s")


def _edge_scalar_body(as_hbm, ad_hbm, srcs_hbm, dsts_hbm, ex_hbm, den_hbm,
                      as_v, ad_v, src_v, dst_v, ex_v, den_v):
    cid = lax.axis_index("c")
    sid = lax.axis_index("s")
    wid = sid * 2 + cid
    pltpu.sync_copy(as_hbm, as_v)
    pltpu.sync_copy(ad_hbm, ad_v)
    pltpu.sync_copy(srcs_hbm.at[pl.ds(wid * RA, RA)], src_v)
    pltpu.sync_copy(dsts_hbm.at[pl.ds(wid * RA, RA)], dst_v)

    def zero_body(i, _):
        den_v[pl.ds(i * 16, 16)] = jnp.zeros((16,), jnp.float32)
        return 0
    lax.fori_loop(0, NPAD // 16, zero_body, 0)

    def row_body(r, _):
        def col_body(k, _):
            sl = pl.ds(k * 16, 16)
            src16 = src_v[r, sl]
            dst16 = dst_v[r, sl]
            asg = plsc.load_gather(as_v, [src16])
            adg = plsc.load_gather(ad_v, [dst16])
            e = asg + adg
            e = jnp.maximum(e, e * 0.2)
            ex = jnp.exp(e)
            ex_v[r, sl] = ex
            plsc.addupdate_scatter(den_v, [dst16], ex)
            return 0
        lax.fori_loop(0, C // 16, col_body, 0, unroll=True)
        return 0
    lax.fori_loop(0, RA, row_body, 0)

    pltpu.sync_copy(ex_v, ex_hbm.at[pl.ds(wid * RA, RA)])
    pltpu.sync_copy(den_v, den_hbm.at[pl.ds(wid * NPAD, NPAD)])


def _edge_scalars(a_src, a_dst, srcs2d, dsts2d):
    k = pl.kernel(
        _edge_scalar_body,
        out_type=(
            jax.ShapeDtypeStruct((EROWS, C), jnp.float32),
            jax.ShapeDtypeStruct((32 * NPAD,), jnp.float32),
        ),
        mesh=_sc_mesh(),
        compiler_params=_SC_PARAMS,
        scratch_types=[
            pltpu.VMEM((NPAD,), jnp.float32),
            pltpu.VMEM((NPAD,), jnp.float32),
            pltpu.VMEM((RA, C), jnp.int32),
            pltpu.VMEM((RA, C), jnp.int32),
            pltpu.VMEM((RA, C), jnp.float32),
            pltpu.VMEM((NPAD,), jnp.float32),
        ],
    )
    return k(a_src, a_dst, srcs2d, dsts2d)


def _edge_agg_body(hL_hbm, hR_hbm, srcs_hbm, dsts_hbm, exf_hbm,
                   numL_hbm, numR_hbm,
                   src_v, dst_v, ex_v, rows0_v, rows1_v, acc_sh,
                   gsem0, gsem1, ssem0, ssem1):
    cid = lax.axis_index("c")
    sid = lax.axis_index("s")
    rows = (rows0_v, rows1_v)
    gsems = (gsem0, gsem1)
    ssems = (ssem0, ssem1)

    def zrow(i, _):
        def zcol(j, _):
            rows0_v[i, pl.ds(j * 16, 16)] = jnp.zeros((16,), jnp.float32)
            return 0
        lax.fori_loop(0, HH // 16, zcol, 0, unroll=True)
        return 0
    lax.fori_loop(0, C, zrow, 0)
    for j in range(NPT // C):
        pltpu.sync_copy(rows0_v, acc_sh.at[pl.ds(sid * NPT + j * C, C)])
    plsc.subcore_barrier()

    def start_gather(c, b):
        @pl.when(cid == 0)
        def _():
            pltpu.async_copy(hL_hbm.at[src_v.at[c]], rows[b], gsems[b])

        @pl.when(cid == 1)
        def _():
            pltpu.async_copy(hR_hbm.at[src_v.at[c]], rows[b], gsems[b])

    def wait_gather(b):
        pltpu.make_async_copy(hL_hbm.at[pl.ds(0, C)], rows[b], gsems[b]).wait()

    def start_scatter(c, b):
        pltpu.async_copy(rows[b], acc_sh.at[dst_v.at[c]], ssems[b], add=True)

    def wait_scatter(b):
        pltpu.make_async_copy(rows[b], acc_sh.at[pl.ds(0, C)], ssems[b]).wait()

    def scale(c, b):
        def edge_body(i, _):
            exv = plsc.load_gather(
                ex_v, [jnp.zeros((16,), jnp.int32) + (c * C + i)])

            def col_body(j, _):
                sl = pl.ds(j * 16, 16)
                rows[b][i, sl] = rows[b][i, sl] * exv
                return 0
            lax.fori_loop(0, HH // 16, col_body, 0, unroll=True)
            return 0
        lax.fori_loop(0, C, edge_body, 0)

    def group_body(g, _):
        row0 = sid * RB + g * GR
        pltpu.sync_copy(srcs_hbm.at[pl.ds(row0, GR)], src_v)
        pltpu.sync_copy(dsts_hbm.at[pl.ds(row0, GR)], dst_v)
        pltpu.sync_copy(exf_hbm.at[pl.ds(row0 * C, GR * C)], ex_v)
        start_gather(0, 0)
        for c in range(GR):
            b = c % 2
            if c + 1 < GR:
                if c >= 1:
                    wait_scatter(1 - b)
                start_gather(c + 1, 1 - b)
            wait_gather(b)
            scale(c, b)
            start_scatter(c, b)
        wait_scatter(0)
        wait_scatter(1)
        return 0
    lax.fori_loop(0, RB // GR, group_body, 0)
    plsc.subcore_barrier()

    @pl.when(cid == 0)
    def _():
        pltpu.sync_copy(acc_sh.at[pl.ds(sid * NPT, NPT)],
                        numL_hbm.at[pl.ds(sid * NPT, NPT)])

    @pl.when(cid == 1)
    def _():
        pltpu.sync_copy(acc_sh.at[pl.ds(sid * NPT, NPT)],
                        numR_hbm.at[pl.ds(sid * NPT, NPT)])


def _edge_aggregate(hL, hR, srcs2d, dsts2d, ex_flat):
    k = pl.kernel(
        _edge_agg_body,
        out_type=(
            jax.ShapeDtypeStruct((NPAD, HH), jnp.float32),
            jax.ShapeDtypeStruct((NPAD, HH), jnp.float32),
        ),
        mesh=_sc_mesh(),
        compiler_params=_SC_PARAMS,
        scratch_types=[
            pltpu.VMEM((GR, C), jnp.int32),
            pltpu.VMEM((GR, C), jnp.int32),
            pltpu.VMEM((GR * C,), jnp.float32),
            pltpu.VMEM((C, HH), jnp.float32),
            pltpu.VMEM((C, HH), jnp.float32),
            pltpu.VMEM_SHARED((NPAD, HH), jnp.float32),
            pltpu.SemaphoreType.DMA,
            pltpu.SemaphoreType.DMA,
            pltpu.SemaphoreType.DMA,
            pltpu.SemaphoreType.DMA,
        ],
    )
    return k(hL, hR, srcs2d, dsts2d, ex_flat)


def _mm_body(x_ref, w_ref, a2_ref, hL_ref, hR_ref, sc_ref):
    h = jnp.dot(x_ref[...], w_ref[...], preferred_element_type=jnp.float32)
    hL_ref[...] = h[:, :HH]
    hR_ref[...] = h[:, HH:]
    sc_ref[...] = jnp.dot(h, a2_ref[...], preferred_element_type=jnp.float32)


def _matmul_scores(x, W, aS, aD):
    K = x.shape[1]
    A2 = jnp.zeros((H, 8), jnp.float32).at[:, 0].set(aS).at[:, 1].set(aD)
    out = pl.pallas_call(
        _mm_body,
        grid=(NBLK,),
        in_specs=[
            pl.BlockSpec((256, K), lambda i: (i, 0)),
            pl.BlockSpec((K, H), lambda i: (0, 0)),
            pl.BlockSpec((H, 8), lambda i: (0, 0)),
        ],
        out_specs=(
            pl.BlockSpec((256, HH), lambda i: (i, 0)),
            pl.BlockSpec((256, HH), lambda i: (i, 0)),
            pl.BlockSpec((256, 8), lambda i: (i, 0)),
        ),
        out_shape=(
            jax.ShapeDtypeStruct((NPAD, HH), jnp.float32),
            jax.ShapeDtypeStruct((NPAD, HH), jnp.float32),
            jax.ShapeDtypeStruct((NPAD, 8), jnp.float32),
        ),
    )(x, W, A2)
    return out


def _onehot(bt):
    return (bt[None, :] == lax.broadcasted_iota(jnp.int32, (G, 256), 0)
            ).astype(jnp.float32)


def _post1_body(numL_ref, numR_ref, den_ref, b_ref, bt_ref,
                y_ref, gsum_ref, cnt_ref):
    i = pl.program_id(0)
    den = jnp.sum(den_ref[...], axis=0) + 1e-16
    num = jnp.concatenate([numL_ref[...], numR_ref[...]], axis=1)
    y = num / den[:, None] + b_ref[...][None, :]
    y = jnp.maximum(y, 0.01 * y)
    y_ref[...] = y
    oh = _onehot(bt_ref[0, 0, :])
    part = jnp.dot(oh, y, preferred_element_type=jnp.float32)
    pcnt = jnp.dot(oh, jnp.ones((256, 8), jnp.float32),
                   preferred_element_type=jnp.float32)

    @pl.when(i == 0)
    def _():
        gsum_ref[...] = jnp.zeros_like(gsum_ref)
        cnt_ref[...] = jnp.zeros_like(cnt_ref)

    gsum_ref[...] += part
    cnt_ref[...] += pcnt


def _post1(numL, numR, den_parts, b, batch3):
    return pl.pallas_call(
        _post1_body,
        grid=(NBLK,),
        in_specs=[
            pl.BlockSpec((256, HH), lambda i: (i, 0)),
            pl.BlockSpec((256, HH), lambda i: (i, 0)),
            pl.BlockSpec((32, 256), lambda i: (0, i)),
            pl.BlockSpec((H,), lambda i: (0,)),
            pl.BlockSpec((1, 1, 256), lambda i: (i, 0, 0)),
        ],
        out_specs=(
            pl.BlockSpec((256, H), lambda i: (i, 0)),
            pl.BlockSpec((G, H), lambda i: (0, 0)),
            pl.BlockSpec((G, 8), lambda i: (0, 0)),
        ),
        out_shape=(
            jax.ShapeDtypeStruct((NPAD, H), jnp.float32),
            jax.ShapeDtypeStruct((G, H), jnp.float32),
            jax.ShapeDtypeStruct((G, 8), jnp.float32),
        ),
    )(numL, numR, den_parts, b, batch3)


def _var_body(y_ref, gsum_ref, cnt_ref, ms_ref, bt_ref, vsum_ref):
    i = pl.program_id(0)
    cnt = jnp.maximum(cnt_ref[...][:, :1], 1.0)
    mean = gsum_ref[...] / cnt
    bt = bt_ref[0, 0, :]
    oh = _onehot(bt)
    mg = jnp.dot(oh.T, mean, preferred_element_type=jnp.float32)
    oc = y_ref[...] - mg * ms_ref[...][None, :]
    part = jnp.dot(oh, oc * oc, preferred_element_type=jnp.float32)

    @pl.when(i == 0)
    def _():
        vsum_ref[...] = jnp.zeros_like(vsum_ref)

    vsum_ref[...] += part


def _var_pass(y, gsum, cnt, ms, batch3):
    return pl.pallas_call(
        _var_body,
        grid=(NBLK,),
        in_specs=[
            pl.BlockSpec((256, H), lambda i: (i, 0)),
            pl.BlockSpec((G, H), lambda i: (0, 0)),
            pl.BlockSpec((G, 8), lambda i: (0, 0)),
            pl.BlockSpec((H,), lambda i: (0,)),
            pl.BlockSpec((1, 1, 256), lambda i: (i, 0, 0)),
        ],
        out_specs=pl.BlockSpec((G, H), lambda i: (0, 0)),
        out_shape=jax.ShapeDtypeStruct((G, H), jnp.float32),
    )(y, gsum, cnt, ms, batch3)


def _norm_body(y_ref, gsum_ref, cnt_ref, vsum_ref, w_ref, bias_ref, ms_ref,
               bt_ref, out_ref):
    cnt = jnp.maximum(cnt_ref[...][:, :1], 1.0)
    mean = gsum_ref[...] / cnt
    std = jnp.sqrt(vsum_ref[...] / cnt + 1e-5)
    bt = bt_ref[0, 0, :]
    oh = _onehot(bt)
    mg = jnp.dot(oh.T, mean, preferred_element_type=jnp.float32)
    sg = jnp.dot(oh.T, std, preferred_element_type=jnp.float32)
    oc = y_ref[...] - mg * ms_ref[...][None, :]
    out = w_ref[...][None, :] * oc / sg + bias_ref[...][None, :]
    out_ref[...] = jnp.where(sg > 0, out, 0.0)


def _norm_pass(y, gsum, cnt, vsum, w, bias, ms, batch3):
    return pl.pallas_call(
        _norm_body,
        grid=(NBLK,),
        in_specs=[
            pl.BlockSpec((256, H), lambda i: (i, 0)),
            pl.BlockSpec((G, H), lambda i: (0, 0)),
            pl.BlockSpec((G, 8), lambda i: (0, 0)),
            pl.BlockSpec((G, H), lambda i: (0, 0)),
            pl.BlockSpec((H,), lambda i: (0,)),
            pl.BlockSpec((H,), lambda i: (0,)),
            pl.BlockSpec((H,), lambda i: (0,)),
            pl.BlockSpec((1, 1, 256), lambda i: (i, 0, 0)),
        ],
        out_specs=pl.BlockSpec((256, H), lambda i: (i, 0)),
        out_shape=jax.ShapeDtypeStruct((NPAD, H), jnp.float32),
    )(y, gsum, cnt, vsum, w, bias, ms, batch3)


def _post2_body(numL_ref, numR_ref, den_ref, b_ref, out_ref):
    den = jnp.sum(den_ref[...], axis=0) + 1e-16
    num = jnp.concatenate([numL_ref[...], numR_ref[...]], axis=1)
    y = num / den[:, None] + b_ref[...][None, :]
    out_ref[...] = jnp.maximum(y, 0.01 * y)


def _post2(numL, numR, den_parts, b):
    return pl.pallas_call(
        _post2_body,
        grid=(NBLK,),
        in_specs=[
            pl.BlockSpec((256, HH), lambda i: (i, 0)),
            pl.BlockSpec((256, HH), lambda i: (i, 0)),
            pl.BlockSpec((32, 256), lambda i: (0, i)),
            pl.BlockSpec((H,), lambda i: (0,)),
        ],
        out_specs=pl.BlockSpec((256, H), lambda i: (i, 0)),
        out_shape=jax.ShapeDtypeStruct((NPAD, H), jnp.float32),
    )(numL, numR, den_parts, b)


def _post3_body(h2_ref, numL_ref, numR_ref, den_ref, b_ref, cnt_ref, bt_ref,
                pool_ref):
    i = pl.program_id(0)
    den = jnp.sum(den_ref[...], axis=0) + 1e-16
    num = jnp.concatenate([numL_ref[...], numR_ref[...]], axis=1)
    y = h2_ref[...] + num / den[:, None] + b_ref[...][None, :]
    oh = _onehot(bt_ref[0, 0, :])
    part = jnp.dot(oh, y, preferred_element_type=jnp.float32)

    @pl.when(i == 0)
    def _():
        pool_ref[...] = jnp.zeros_like(pool_ref)

    pool_ref[...] += part

    @pl.when(i == NBLK - 1)
    def _():
        cnt = jnp.maximum(cnt_ref[...][:, :1], 1.0)
        pool_ref[...] = pool_ref[...] / cnt


def _post3_pool(h2, numL, numR, den_parts, b, cnt, batch3):
    return pl.pallas_call(
        _post3_body,
        grid=(NBLK,),
        in_specs=[
            pl.BlockSpec((256, H), lambda i: (i, 0)),
            pl.BlockSpec((256, HH), lambda i: (i, 0)),
            pl.BlockSpec((256, HH), lambda i: (i, 0)),
            pl.BlockSpec((32, 256), lambda i: (0, i)),
            pl.BlockSpec((H,), lambda i: (0,)),
            pl.BlockSpec((G, 8), lambda i: (0, 0)),
            pl.BlockSpec((1, 1, 256), lambda i: (i, 0, 0)),
        ],
        out_specs=pl.BlockSpec((G, H), lambda i: (0, 0)),
        out_shape=jax.ShapeDtypeStruct((G, H), jnp.float32),
    )(h2, numL, numR, den_parts, b, cnt, batch3)


def _gat_layer(h, W, aS, aD, srcs2d, dsts2d):
    hL, hR, sc8 = _matmul_scores(h, W, aS, aD)
    a_src = sc8[:, 0]
    a_dst = sc8[:, 1]
    ex2d, den_flat = _edge_scalars(a_src, a_dst, srcs2d, dsts2d)
    numL, numR = _edge_aggregate(hL, hR, srcs2d, dsts2d,
                                 ex2d.reshape(EPAD))
    return numL, numR, den_flat.reshape(32, NPAD)


def kernel(x, edge_index, batch, W1, aS1, aD1, b1, W2, aS2, aD2, b2,
           W3, aS3, aD3, b3, gn_weight, gn_bias, gn_mean_scale):
    loop = jnp.arange(N, dtype=jnp.int32)
    srcs = jnp.concatenate([edge_index[0], loop,
                            jnp.full((EPAD - ETOT,), N, jnp.int32)])
    dsts = jnp.concatenate([edge_index[1], loop,
                            jnp.full((EPAD - ETOT,), N, jnp.int32)])
    key = jnp.sort((srcs << 14) | dsts)
    srcs = key >> 14
    dsts = key & jnp.int32(16383)
    srcs2d = srcs.reshape(EROWS, C)
    dsts2d = dsts.reshape(EROWS, C)
    batch_pad = jnp.concatenate([batch, jnp.full((NPAD - N,), G, jnp.int32)])
    batch3 = batch_pad.reshape(NBLK, 1, 256)
    x_pad = jnp.pad(x, ((0, NPAD - N), (0, 0)))

    numL, numR, den = _gat_layer(x_pad, W1, aS1, aD1, srcs2d, dsts2d)
    y1, gsum, cnt = _post1(numL, numR, den, b1, batch3)
    vsum = _var_pass(y1, gsum, cnt, gn_mean_scale, batch3)
    h1 = _norm_pass(y1, gsum, cnt, vsum, gn_weight, gn_bias,
                    gn_mean_scale, batch3)

    numL, numR, den = _gat_layer(h1, W2, aS2, aD2, srcs2d, dsts2d)
    h2 = _post2(numL, numR, den, b2)

    numL, numR, den = _gat_layer(h2, W3, aS3, aD3, srcs2d, dsts2d)
    return _post3_pool(h2, numL, numR, den, b3, cnt, batch3)


# speedup vs baseline: 1.8193x; 1.8193x over previous
import functools

import jax
import jax.numpy as jnp
from jax import lax
from jax.experimental import pallas as pl
from jax.experimental.pallas import tpu as pltpu
from jax.experimental.pallas import tpu_sc as plsc

N = 10000
E = 320000
IN = 128
H = 256
G = 64

NPAD = 10240
HH = H // 2
ETOT = E + N
C = 128
EROWS = 2688
EPAD = EROWS * C
EPT_A = EPAD // 32
RB = EROWS // 16
GR = 8
NBLK = NPAD // 256
NPT = NPAD // 16

_SC_PARAMS = pltpu.CompilerParams(needs_layout_passes=False)


def _sc_mesh():
    return plsc.VectorSubcoreMesh(core_axis_name="c", subcore_axis_name="s")


def _edge_scalar_body(as_hbm, ad_hbm, srcsf_hbm, dstsf_hbm, ex_hbm, den_hbm,
                      as_v, ad_v, src_v, dst_v, ex_v, den_v):
    cid = lax.axis_index("c")
    sid = lax.axis_index("s")
    wid = sid * 2 + cid
    pltpu.sync_copy(as_hbm, as_v)
    pltpu.sync_copy(ad_hbm, ad_v)
    pltpu.sync_copy(srcsf_hbm.at[pl.ds(wid * EPT_A, EPT_A)], src_v)
    pltpu.sync_copy(dstsf_hbm.at[pl.ds(wid * EPT_A, EPT_A)], dst_v)

    def zero_body(i, _):
        den_v[pl.ds(i * 16, 16)] = jnp.zeros((16,), jnp.float32)
        return 0
    lax.fori_loop(0, NPAD // 16, zero_body, 0)

    def edge16_body(e, _):
        sl = pl.ds(e * 16, 16)
        src16 = src_v[sl]
        dst16 = dst_v[sl]
        asg = plsc.load_gather(as_v, [src16])
        adg = plsc.load_gather(ad_v, [dst16])
        ee = asg + adg
        ee = jnp.maximum(ee, ee * 0.2)
        ex = jnp.exp(ee)
        ex_v[sl] = ex
        plsc.addupdate_scatter(den_v, [dst16], ex)
        return 0
    lax.fori_loop(0, EPT_A // 16, edge16_body, 0)

    pltpu.sync_copy(ex_v, ex_hbm.at[pl.ds(wid * EPT_A, EPT_A)])
    pltpu.sync_copy(den_v, den_hbm.at[pl.ds(wid * NPAD, NPAD)])


def _edge_scalars(a_src, a_dst, srcs_flat, dsts_flat):
    k = pl.kernel(
        _edge_scalar_body,
        out_type=(
            jax.ShapeDtypeStruct((EPAD,), jnp.float32),
            jax.ShapeDtypeStruct((32 * NPAD,), jnp.float32),
        ),
        mesh=_sc_mesh(),
        compiler_params=_SC_PARAMS,
        scratch_types=[
            pltpu.VMEM((NPAD,), jnp.float32),
            pltpu.VMEM((NPAD,), jnp.float32),
            pltpu.VMEM((EPT_A,), jnp.int32),
            pltpu.VMEM((EPT_A,), jnp.int32),
            pltpu.VMEM((EPT_A,), jnp.float32),
            pltpu.VMEM((NPAD,), jnp.float32),
        ],
    )
    return k(a_src, a_dst, srcs_flat, dsts_flat)


def _edge_agg_body(hL_hbm, hR_hbm, srcs_hbm, dsts_hbm, exf_hbm,
                   numL_hbm, numR_hbm,
                   src_v, dst_v, ex_v, rows0_v, rows1_v, acc_sh,
                   gsem0, gsem1, ssem0, ssem1):
    cid = lax.axis_index("c")
    sid = lax.axis_index("s")
    rows = (rows0_v, rows1_v)
    gsems = (gsem0, gsem1)
    ssems = (ssem0, ssem1)

    def zrow(i, _):
        def zcol(j, _):
            rows0_v[i, pl.ds(j * 16, 16)] = jnp.zeros((16,), jnp.float32)
            return 0
        lax.fori_loop(0, HH // 16, zcol, 0, unroll=True)
        return 0
    lax.fori_loop(0, C, zrow, 0)
    for j in range(NPT // C):
        pltpu.sync_copy(rows0_v, acc_sh.at[pl.ds(sid * NPT + j * C, C)])
    plsc.subcore_barrier()

    def start_gather(c, b):
        @pl.when(cid == 0)
        def _():
            pltpu.async_copy(hL_hbm.at[src_v.at[c]], rows[b], gsems[b])

        @pl.when(cid == 1)
        def _():
            pltpu.async_copy(hR_hbm.at[src_v.at[c]], rows[b], gsems[b])

    def wait_gather(b):
        pltpu.make_async_copy(hL_hbm.at[pl.ds(0, C)], rows[b], gsems[b]).wait()

    def start_scatter(c, b):
        pltpu.async_copy(rows[b], acc_sh.at[dst_v.at[c]], ssems[b], add=True)

    def wait_scatter(b):
        pltpu.make_async_copy(rows[b], acc_sh.at[pl.ds(0, C)], ssems[b]).wait()

    def scale(c, b):
        def edge_body(i, _):
            exv = plsc.load_gather(
                ex_v, [jnp.zeros((16,), jnp.int32) + (c * C + i)])

            def col_body(j, _):
                sl = pl.ds(j * 16, 16)
                rows[b][i, sl] = rows[b][i, sl] * exv
                return 0
            lax.fori_loop(0, HH // 16, col_body, 0, unroll=True)
            return 0
        lax.fori_loop(0, C, edge_body, 0)

    def group_body(g, _):
        row0 = sid * RB + g * GR
        pltpu.sync_copy(srcs_hbm.at[pl.ds(row0, GR)], src_v)
        pltpu.sync_copy(dsts_hbm.at[pl.ds(row0, GR)], dst_v)
        pltpu.sync_copy(exf_hbm.at[pl.ds(row0 * C, GR * C)], ex_v)
        start_gather(0, 0)
        for c in range(GR):
            b = c % 2
            if c + 1 < GR:
                if c >= 1:
                    wait_scatter(1 - b)
                start_gather(c + 1, 1 - b)
            wait_gather(b)
            scale(c, b)
            start_scatter(c, b)
        wait_scatter(0)
        wait_scatter(1)
        return 0
    lax.fori_loop(0, RB // GR, group_body, 0)
    plsc.subcore_barrier()

    @pl.when(cid == 0)
    def _():
        pltpu.sync_copy(acc_sh.at[pl.ds(sid * NPT, NPT)],
                        numL_hbm.at[pl.ds(sid * NPT, NPT)])

    @pl.when(cid == 1)
    def _():
        pltpu.sync_copy(acc_sh.at[pl.ds(sid * NPT, NPT)],
                        numR_hbm.at[pl.ds(sid * NPT, NPT)])


def _edge_aggregate(hL, hR, srcs2d, dsts2d, ex_flat):
    k = pl.kernel(
        _edge_agg_body,
        out_type=(
            jax.ShapeDtypeStruct((NPAD, HH), jnp.float32),
            jax.ShapeDtypeStruct((NPAD, HH), jnp.float32),
        ),
        mesh=_sc_mesh(),
        compiler_params=_SC_PARAMS,
        scratch_types=[
            pltpu.VMEM((GR, C), jnp.int32),
            pltpu.VMEM((GR, C), jnp.int32),
            pltpu.VMEM((GR * C,), jnp.float32),
            pltpu.VMEM((C, HH), jnp.float32),
            pltpu.VMEM((C, HH), jnp.float32),
            pltpu.VMEM_SHARED((NPAD, HH), jnp.float32),
            pltpu.SemaphoreType.DMA,
            pltpu.SemaphoreType.DMA,
            pltpu.SemaphoreType.DMA,
            pltpu.SemaphoreType.DMA,
        ],
    )
    return k(hL, hR, srcs2d, dsts2d, ex_flat)


def _mm_body(x_ref, w_ref, a2_ref, hL_ref, hR_ref, sc_ref):
    h = jnp.dot(x_ref[...], w_ref[...], preferred_element_type=jnp.float32)
    hL_ref[...] = h[:, :HH]
    hR_ref[...] = h[:, HH:]
    sc_ref[...] = jnp.dot(h, a2_ref[...], preferred_element_type=jnp.float32)


def _matmul_scores(x, W, aS, aD):
    K = x.shape[1]
    A2 = jnp.zeros((H, 8), jnp.float32).at[:, 0].set(aS).at[:, 1].set(aD)
    out = pl.pallas_call(
        _mm_body,
        grid=(NBLK,),
        in_specs=[
            pl.BlockSpec((256, K), lambda i: (i, 0)),
            pl.BlockSpec((K, H), lambda i: (0, 0)),
            pl.BlockSpec((H, 8), lambda i: (0, 0)),
        ],
        out_specs=(
            pl.BlockSpec((256, HH), lambda i: (i, 0)),
            pl.BlockSpec((256, HH), lambda i: (i, 0)),
            pl.BlockSpec((256, 8), lambda i: (i, 0)),
        ),
        out_shape=(
            jax.ShapeDtypeStruct((NPAD, HH), jnp.float32),
            jax.ShapeDtypeStruct((NPAD, HH), jnp.float32),
            jax.ShapeDtypeStruct((NPAD, 8), jnp.float32),
        ),
    )(x, W, A2)
    return out


def _onehot(bt):
    return (bt[None, :] == lax.broadcasted_iota(jnp.int32, (G, 256), 0)
            ).astype(jnp.float32)


def _post1_body(numL_ref, numR_ref, den_ref, b_ref, bt_ref,
                y_ref, gsum_ref, cnt_ref):
    i = pl.program_id(0)
    den = jnp.sum(den_ref[...], axis=0) + 1e-16
    num = jnp.concatenate([numL_ref[...], numR_ref[...]], axis=1)
    y = num / den[:, None] + b_ref[...][None, :]
    y = jnp.maximum(y, 0.01 * y)
    y_ref[...] = y
    oh = _onehot(bt_ref[0, 0, :])
    part = jnp.dot(oh, y, preferred_element_type=jnp.float32)
    pcnt = jnp.dot(oh, jnp.ones((256, 8), jnp.float32),
                   preferred_element_type=jnp.float32)

    @pl.when(i == 0)
    def _():
        gsum_ref[...] = jnp.zeros_like(gsum_ref)
        cnt_ref[...] = jnp.zeros_like(cnt_ref)

    gsum_ref[...] += part
    cnt_ref[...] += pcnt


def _post1(numL, numR, den_parts, b, batch3):
    return pl.pallas_call(
        _post1_body,
        grid=(NBLK,),
        in_specs=[
            pl.BlockSpec((256, HH), lambda i: (i, 0)),
            pl.BlockSpec((256, HH), lambda i: (i, 0)),
            pl.BlockSpec((32, 256), lambda i: (0, i)),
            pl.BlockSpec((H,), lambda i: (0,)),
            pl.BlockSpec((1, 1, 256), lambda i: (i, 0, 0)),
        ],
        out_specs=(
            pl.BlockSpec((256, H), lambda i: (i, 0)),
            pl.BlockSpec((G, H), lambda i: (0, 0)),
            pl.BlockSpec((G, 8), lambda i: (0, 0)),
        ),
        out_shape=(
            jax.ShapeDtypeStruct((NPAD, H), jnp.float32),
            jax.ShapeDtypeStruct((G, H), jnp.float32),
            jax.ShapeDtypeStruct((G, 8), jnp.float32),
        ),
    )(numL, numR, den_parts, b, batch3)


def _var_body(y_ref, gsum_ref, cnt_ref, ms_ref, bt_ref, vsum_ref):
    i = pl.program_id(0)
    cnt = jnp.maximum(cnt_ref[...][:, :1], 1.0)
    mean = gsum_ref[...] / cnt
    bt = bt_ref[0, 0, :]
    oh = _onehot(bt)
    mg = jnp.dot(oh.T, mean, preferred_element_type=jnp.float32)
    oc = y_ref[...] - mg * ms_ref[...][None, :]
    part = jnp.dot(oh, oc * oc, preferred_element_type=jnp.float32)

    @pl.when(i == 0)
    def _():
        vsum_ref[...] = jnp.zeros_like(vsum_ref)

    vsum_ref[...] += part


def _var_pass(y, gsum, cnt, ms, batch3):
    return pl.pallas_call(
        _var_body,
        grid=(NBLK,),
        in_specs=[
            pl.BlockSpec((256, H), lambda i: (i, 0)),
            pl.BlockSpec((G, H), lambda i: (0, 0)),
            pl.BlockSpec((G, 8), lambda i: (0, 0)),
            pl.BlockSpec((H,), lambda i: (0,)),
            pl.BlockSpec((1, 1, 256), lambda i: (i, 0, 0)),
        ],
        out_specs=pl.BlockSpec((G, H), lambda i: (0, 0)),
        out_shape=jax.ShapeDtypeStruct((G, H), jnp.float32),
    )(y, gsum, cnt, ms, batch3)


def _norm_body(y_ref, gsum_ref, cnt_ref, vsum_ref, w_ref, bias_ref, ms_ref,
               bt_ref, out_ref):
    cnt = jnp.maximum(cnt_ref[...][:, :1], 1.0)
    mean = gsum_ref[...] / cnt
    std = jnp.sqrt(vsum_ref[...] / cnt + 1e-5)
    bt = bt_ref[0, 0, :]
    oh = _onehot(bt)
    mg = jnp.dot(oh.T, mean, preferred_element_type=jnp.float32)
    sg = jnp.dot(oh.T, std, preferred_element_type=jnp.float32)
    oc = y_ref[...] - mg * ms_ref[...][None, :]
    out = w_ref[...][None, :] * oc / sg + bias_ref[...][None, :]
    out_ref[...] = jnp.where(sg > 0, out, 0.0)


def _norm_pass(y, gsum, cnt, vsum, w, bias, ms, batch3):
    return pl.pallas_call(
        _norm_body,
        grid=(NBLK,),
        in_specs=[
            pl.BlockSpec((256, H), lambda i: (i, 0)),
            pl.BlockSpec((G, H), lambda i: (0, 0)),
            pl.BlockSpec((G, 8), lambda i: (0, 0)),
            pl.BlockSpec((G, H), lambda i: (0, 0)),
            pl.BlockSpec((H,), lambda i: (0,)),
            pl.BlockSpec((H,), lambda i: (0,)),
            pl.BlockSpec((H,), lambda i: (0,)),
            pl.BlockSpec((1, 1, 256), lambda i: (i, 0, 0)),
        ],
        out_specs=pl.BlockSpec((256, H), lambda i: (i, 0)),
        out_shape=jax.ShapeDtypeStruct((NPAD, H), jnp.float32),
    )(y, gsum, cnt, vsum, w, bias, ms, batch3)


def _post2_body(numL_ref, numR_ref, den_ref, b_ref, out_ref):
    den = jnp.sum(den_ref[...], axis=0) + 1e-16
    num = jnp.concatenate([numL_ref[...], numR_ref[...]], axis=1)
    y = num / den[:, None] + b_ref[...][None, :]
    out_ref[...] = jnp.maximum(y, 0.01 * y)


def _post2(numL, numR, den_parts, b):
    return pl.pallas_call(
        _post2_body,
        grid=(NBLK,),
        in_specs=[
            pl.BlockSpec((256, HH), lambda i: (i, 0)),
            pl.BlockSpec((256, HH), lambda i: (i, 0)),
            pl.BlockSpec((32, 256), lambda i: (0, i)),
            pl.BlockSpec((H,), lambda i: (0,)),
        ],
        out_specs=pl.BlockSpec((256, H), lambda i: (i, 0)),
        out_shape=jax.ShapeDtypeStruct((NPAD, H), jnp.float32),
    )(numL, numR, den_parts, b)


def _post3_body(h2_ref, numL_ref, numR_ref, den_ref, b_ref, cnt_ref, bt_ref,
                pool_ref):
    i = pl.program_id(0)
    den = jnp.sum(den_ref[...], axis=0) + 1e-16
    num = jnp.concatenate([numL_ref[...], numR_ref[...]], axis=1)
    y = h2_ref[...] + num / den[:, None] + b_ref[...][None, :]
    oh = _onehot(bt_ref[0, 0, :])
    part = jnp.dot(oh, y, preferred_element_type=jnp.float32)

    @pl.when(i == 0)
    def _():
        pool_ref[...] = jnp.zeros_like(pool_ref)

    pool_ref[...] += part

    @pl.when(i == NBLK - 1)
    def _():
        cnt = jnp.maximum(cnt_ref[...][:, :1], 1.0)
        pool_ref[...] = pool_ref[...] / cnt


def _post3_pool(h2, numL, numR, den_parts, b, cnt, batch3):
    return pl.pallas_call(
        _post3_body,
        grid=(NBLK,),
        in_specs=[
            pl.BlockSpec((256, H), lambda i: (i, 0)),
            pl.BlockSpec((256, HH), lambda i: (i, 0)),
            pl.BlockSpec((256, HH), lambda i: (i, 0)),
            pl.BlockSpec((32, 256), lambda i: (0, i)),
            pl.BlockSpec((H,), lambda i: (0,)),
            pl.BlockSpec((G, 8), lambda i: (0, 0)),
            pl.BlockSpec((1, 1, 256), lambda i: (i, 0, 0)),
        ],
        out_specs=pl.BlockSpec((G, H), lambda i: (0, 0)),
        out_shape=jax.ShapeDtypeStruct((G, H), jnp.float32),
    )(h2, numL, numR, den_parts, b, cnt, batch3)


def _gat_layer(h, W, aS, aD, srcs, dsts, srcs2d, dsts2d):
    hL, hR, sc8 = _matmul_scores(h, W, aS, aD)
    a_src = sc8[:, 0]
    a_dst = sc8[:, 1]
    ex_flat, den_flat = _edge_scalars(a_src, a_dst, srcs, dsts)
    numL, numR = _edge_aggregate(hL, hR, srcs2d, dsts2d, ex_flat)
    return numL, numR, den_flat.reshape(32, NPAD)


def kernel(x, edge_index, batch, W1, aS1, aD1, b1, W2, aS2, aD2, b2,
           W3, aS3, aD3, b3, gn_weight, gn_bias, gn_mean_scale):
    loop = jnp.arange(N, dtype=jnp.int32)
    srcs = jnp.concatenate([edge_index[0], loop,
                            jnp.full((EPAD - ETOT,), N, jnp.int32)])
    dsts = jnp.concatenate([edge_index[1], loop,
                            jnp.full((EPAD - ETOT,), N, jnp.int32)])
    srcs2d = srcs.reshape(EROWS, C)
    dsts2d = dsts.reshape(EROWS, C)
    batch_pad = jnp.concatenate([batch, jnp.full((NPAD - N,), G, jnp.int32)])
    batch3 = batch_pad.reshape(NBLK, 1, 256)
    x_pad = jnp.pad(x, ((0, NPAD - N), (0, 0)))

    numL, numR, den = _gat_layer(x_pad, W1, aS1, aD1, srcs, dsts, srcs2d, dsts2d)
    y1, gsum, cnt = _post1(numL, numR, den, b1, batch3)
    vsum = _var_pass(y1, gsum, cnt, gn_mean_scale, batch3)
    h1 = _norm_pass(y1, gsum, cnt, vsum, gn_weight, gn_bias,
                    gn_mean_scale, batch3)

    numL, numR, den = _gat_layer(h1, W2, aS2, aD2, srcs, dsts, srcs2d, dsts2d)
    h2 = _post2(numL, numR, den, b2)

    numL, numR, den = _gat_layer(h2, W3, aS3, aD3, srcs, dsts, srcs2d, dsts2d)
    return _post3_pool(h2, numL, numR, den, b3, cnt, batch3)


# speedup vs baseline: 3.8559x; 2.1194x over previous
import functools

import jax
import jax.numpy as jnp
from jax import lax
from jax.experimental import pallas as pl
from jax.experimental.pallas import tpu as pltpu
from jax.experimental.pallas import tpu_sc as plsc

N = 10000
E = 320000
IN = 128
H = 256
G = 64

NPAD = 10240
HH = H // 2
ETOT = E + N
C = 128
EROWS = 2688
EPAD = EROWS * C
EPT_A = EPAD // 32
RB = EROWS // 16
GR = 8
NBLK = NPAD // 256
NPT = NPAD // 16

_SC_PARAMS = pltpu.CompilerParams(needs_layout_passes=False)


def _sc_mesh():
    return plsc.VectorSubcoreMesh(core_axis_name="c", subcore_axis_name="s")


def _edge_scalar_body(as_hbm, ad_hbm, srcsf_hbm, dstsf_hbm, ex_hbm, den_hbm,
                      as_v, ad_v, src_v, dst_v, ex_v, den_v):
    cid = lax.axis_index("c")
    sid = lax.axis_index("s")
    wid = sid * 2 + cid
    pltpu.sync_copy(as_hbm, as_v)
    pltpu.sync_copy(ad_hbm, ad_v)
    pltpu.sync_copy(srcsf_hbm.at[pl.ds(wid * EPT_A, EPT_A)], src_v)
    pltpu.sync_copy(dstsf_hbm.at[pl.ds(wid * EPT_A, EPT_A)], dst_v)

    def zero_body(i, _):
        den_v[pl.ds(i * 16, 16)] = jnp.zeros((16,), jnp.float32)
        return 0
    lax.fori_loop(0, NPAD // 16, zero_body, 0)

    def edge16_body(e, _):
        sl = pl.ds(e * 16, 16)
        src16 = src_v[sl]
        dst16 = dst_v[sl]
        asg = plsc.load_gather(as_v, [src16])
        adg = plsc.load_gather(ad_v, [dst16])
        ee = asg + adg
        ee = jnp.maximum(ee, ee * 0.2)
        ex = jnp.exp(ee)
        ex_v[sl] = ex
        plsc.addupdate_scatter(den_v, [dst16], ex)
        return 0
    lax.fori_loop(0, EPT_A // 16, edge16_body, 0)

    pltpu.sync_copy(ex_v, ex_hbm.at[pl.ds(wid * EPT_A, EPT_A)])
    pltpu.sync_copy(den_v, den_hbm.at[pl.ds(wid * NPAD, NPAD)])


def _edge_scalars(a_src, a_dst, srcs_flat, dsts_flat):
    k = pl.kernel(
        _edge_scalar_body,
        out_type=(
            jax.ShapeDtypeStruct((EPAD,), jnp.float32),
            jax.ShapeDtypeStruct((32 * NPAD,), jnp.float32),
        ),
        mesh=_sc_mesh(),
        compiler_params=_SC_PARAMS,
        scratch_types=[
            pltpu.VMEM((NPAD,), jnp.float32),
            pltpu.VMEM((NPAD,), jnp.float32),
            pltpu.VMEM((EPT_A,), jnp.int32),
            pltpu.VMEM((EPT_A,), jnp.int32),
            pltpu.VMEM((EPT_A,), jnp.float32),
            pltpu.VMEM((NPAD,), jnp.float32),
        ],
    )
    return k(a_src, a_dst, srcs_flat, dsts_flat)


def _edge_agg_body(hL_hbm, hR_hbm, srcs_hbm, dsts_hbm, exf_hbm,
                   numL_hbm, numR_hbm,
                   src_v, dst_v, ex_v, rows0_v, rows1_v, acc_sh,
                   gsem0, gsem1, ssem0, ssem1):
    cid = lax.axis_index("c")
    sid = lax.axis_index("s")
    rows = (rows0_v, rows1_v)
    gsems = (gsem0, gsem1)
    ssems = (ssem0, ssem1)

    def zrow(i, _):
        def zcol(j, _):
            rows0_v[i, pl.ds(j * 16, 16)] = jnp.zeros((16,), jnp.float32)
            return 0
        lax.fori_loop(0, HH // 16, zcol, 0, unroll=True)
        return 0
    lax.fori_loop(0, C, zrow, 0)
    for j in range(NPT // C):
        pltpu.sync_copy(rows0_v, acc_sh.at[pl.ds(sid * NPT + j * C, C)])
    plsc.subcore_barrier()

    def start_gather(c, b):
        @pl.when(cid == 0)
        def _():
            pltpu.async_copy(hL_hbm.at[src_v.at[c]], rows[b], gsems[b])

        @pl.when(cid == 1)
        def _():
            pltpu.async_copy(hR_hbm.at[src_v.at[c]], rows[b], gsems[b])

    def wait_gather(b):
        pltpu.make_async_copy(hL_hbm.at[pl.ds(0, C)], rows[b], gsems[b]).wait()

    def start_scatter(c, b):
        pltpu.async_copy(rows[b], acc_sh.at[dst_v.at[c]], ssems[b], add=True)

    def wait_scatter(b):
        pltpu.make_async_copy(rows[b], acc_sh.at[pl.ds(0, C)], ssems[b]).wait()

    def scale(c, b):
        def edge_body(i, _):
            exv = plsc.load_gather(
                ex_v, [jnp.zeros((16,), jnp.int32) + (c * C + i)])

            def col_body(j, _):
                sl = pl.ds(j * 16, 16)
                rows[b][i, sl] = rows[b][i, sl] * exv
                return 0
            lax.fori_loop(0, HH // 16, col_body, 0, unroll=True)
            return 0
        lax.fori_loop(0, C, edge_body, 0)

    def group_body(g, _):
        row0 = sid * RB + g * GR
        pltpu.sync_copy(srcs_hbm.at[pl.ds(row0, GR)], src_v)
        pltpu.sync_copy(dsts_hbm.at[pl.ds(row0, GR)], dst_v)
        pltpu.sync_copy(exf_hbm.at[pl.ds(row0 * C, GR * C)], ex_v)
        start_gather(0, 0)
        for c in range(GR):
            b = c % 2
            if c + 1 < GR:
                if c >= 1:
                    wait_scatter(1 - b)
                start_gather(c + 1, 1 - b)
            wait_gather(b)
            scale(c, b)
            start_scatter(c, b)
        wait_scatter(0)
        wait_scatter(1)
        return 0
    lax.fori_loop(0, RB // GR, group_body, 0)
    plsc.subcore_barrier()

    @pl.when(cid == 0)
    def _():
        pltpu.sync_copy(acc_sh.at[pl.ds(sid * NPT, NPT)],
                        numL_hbm.at[pl.ds(sid * NPT, NPT)])

    @pl.when(cid == 1)
    def _():
        pltpu.sync_copy(acc_sh.at[pl.ds(sid * NPT, NPT)],
                        numR_hbm.at[pl.ds(sid * NPT, NPT)])


def _edge_aggregate(hL, hR, srcs2d, dsts2d, ex_flat):
    k = pl.kernel(
        _edge_agg_body,
        out_type=(
            jax.ShapeDtypeStruct((NPAD, HH), jnp.float32),
            jax.ShapeDtypeStruct((NPAD, HH), jnp.float32),
        ),
        mesh=_sc_mesh(),
        compiler_params=_SC_PARAMS,
        scratch_types=[
            pltpu.VMEM((GR, C), jnp.int32),
            pltpu.VMEM((GR, C), jnp.int32),
            pltpu.VMEM((GR * C,), jnp.float32),
            pltpu.VMEM((C, HH), jnp.float32),
            pltpu.VMEM((C, HH), jnp.float32),
            pltpu.VMEM_SHARED((NPAD, HH), jnp.float32),
            pltpu.SemaphoreType.DMA,
            pltpu.SemaphoreType.DMA,
            pltpu.SemaphoreType.DMA,
            pltpu.SemaphoreType.DMA,
        ],
    )
    return k(hL, hR, srcs2d, dsts2d, ex_flat)


def _mm_body(x_ref, w_ref, a2_ref, hL_ref, hR_ref, sc_ref):
    h = jnp.dot(x_ref[...], w_ref[...], preferred_element_type=jnp.float32)
    hL_ref[...] = h[:, :HH]
    hR_ref[...] = h[:, HH:]
    sc_ref[...] = jnp.dot(h, a2_ref[...], preferred_element_type=jnp.float32)


def _matmul_scores(x, W, aS, aD):
    K = x.shape[1]
    A2 = jnp.zeros((H, 8), jnp.float32).at[:, 0].set(aS).at[:, 1].set(aD)
    out = pl.pallas_call(
        _mm_body,
        grid=(NBLK,),
        in_specs=[
            pl.BlockSpec((256, K), lambda i: (i, 0)),
            pl.BlockSpec((K, H), lambda i: (0, 0)),
            pl.BlockSpec((H, 8), lambda i: (0, 0)),
        ],
        out_specs=(
            pl.BlockSpec((256, HH), lambda i: (i, 0)),
            pl.BlockSpec((256, HH), lambda i: (i, 0)),
            pl.BlockSpec((256, 8), lambda i: (i, 0)),
        ),
        out_shape=(
            jax.ShapeDtypeStruct((NPAD, HH), jnp.float32),
            jax.ShapeDtypeStruct((NPAD, HH), jnp.float32),
            jax.ShapeDtypeStruct((NPAD, 8), jnp.float32),
        ),
    )(x, W, A2)
    return out


def _onehot(bt):
    return (bt[None, :] == lax.broadcasted_iota(jnp.int32, (G, 256), 0)
            ).astype(jnp.float32)


def _post1_body(numL_ref, numR_ref, den_ref, b_ref, bt_ref,
                y_ref, gsum_ref, cnt_ref):
    i = pl.program_id(0)
    den = jnp.sum(den_ref[...], axis=0) + 1e-16
    num = jnp.concatenate([numL_ref[...], numR_ref[...]], axis=1)
    y = num / den[:, None] + b_ref[...][None, :]
    y = jnp.maximum(y, 0.01 * y)
    y_ref[...] = y
    oh = _onehot(bt_ref[0, 0, :])
    part = jnp.dot(oh, y, preferred_element_type=jnp.float32)
    pcnt = jnp.dot(oh, jnp.ones((256, 8), jnp.float32),
                   preferred_element_type=jnp.float32)

    @pl.when(i == 0)
    def _():
        gsum_ref[...] = jnp.zeros_like(gsum_ref)
        cnt_ref[...] = jnp.zeros_like(cnt_ref)

    gsum_ref[...] += part
    cnt_ref[...] += pcnt


def _post1(numL, numR, den_parts, b, batch3):
    return pl.pallas_call(
        _post1_body,
        grid=(NBLK,),
        in_specs=[
            pl.BlockSpec((256, HH), lambda i: (i, 0)),
            pl.BlockSpec((256, HH), lambda i: (i, 0)),
            pl.BlockSpec((32, 256), lambda i: (0, i)),
            pl.BlockSpec((H,), lambda i: (0,)),
            pl.BlockSpec((1, 1, 256), lambda i: (i, 0, 0)),
        ],
        out_specs=(
            pl.BlockSpec((256, H), lambda i: (i, 0)),
            pl.BlockSpec((G, H), lambda i: (0, 0)),
            pl.BlockSpec((G, 8), lambda i: (0, 0)),
        ),
        out_shape=(
            jax.ShapeDtypeStruct((NPAD, H), jnp.float32),
            jax.ShapeDtypeStruct((G, H), jnp.float32),
            jax.ShapeDtypeStruct((G, 8), jnp.float32),
        ),
    )(numL, numR, den_parts, b, batch3)


def _var_body(y_ref, gsum_ref, cnt_ref, ms_ref, bt_ref, vsum_ref):
    i = pl.program_id(0)
    cnt = jnp.maximum(cnt_ref[...][:, :1], 1.0)
    mean = gsum_ref[...] / cnt
    bt = bt_ref[0, 0, :]
    oh = _onehot(bt)
    mg = jnp.dot(oh.T, mean, preferred_element_type=jnp.float32)
    oc = y_ref[...] - mg * ms_ref[...][None, :]
    part = jnp.dot(oh, oc * oc, preferred_element_type=jnp.float32)

    @pl.when(i == 0)
    def _():
        vsum_ref[...] = jnp.zeros_like(vsum_ref)

    vsum_ref[...] += part


def _var_pass(y, gsum, cnt, ms, batch3):
    return pl.pallas_call(
        _var_body,
        grid=(NBLK,),
        in_specs=[
            pl.BlockSpec((256, H), lambda i: (i, 0)),
            pl.BlockSpec((G, H), lambda i: (0, 0)),
            pl.BlockSpec((G, 8), lambda i: (0, 0)),
            pl.BlockSpec((H,), lambda i: (0,)),
            pl.BlockSpec((1, 1, 256), lambda i: (i, 0, 0)),
        ],
        out_specs=pl.BlockSpec((G, H), lambda i: (0, 0)),
        out_shape=jax.ShapeDtypeStruct((G, H), jnp.float32),
    )(y, gsum, cnt, ms, batch3)


def _norm_body(y_ref, gsum_ref, cnt_ref, vsum_ref, w_ref, bias_ref, ms_ref,
               bt_ref, out_ref):
    cnt = jnp.maximum(cnt_ref[...][:, :1], 1.0)
    mean = gsum_ref[...] / cnt
    std = jnp.sqrt(vsum_ref[...] / cnt + 1e-5)
    bt = bt_ref[0, 0, :]
    oh = _onehot(bt)
    mg = jnp.dot(oh.T, mean, preferred_element_type=jnp.float32)
    sg = jnp.dot(oh.T, std, preferred_element_type=jnp.float32)
    oc = y_ref[...] - mg * ms_ref[...][None, :]
    out = w_ref[...][None, :] * oc / sg + bias_ref[...][None, :]
    out_ref[...] = jnp.where(sg > 0, out, 0.0)


def _norm_pass(y, gsum, cnt, vsum, w, bias, ms, batch3):
    return pl.pallas_call(
        _norm_body,
        grid=(NBLK,),
        in_specs=[
            pl.BlockSpec((256, H), lambda i: (i, 0)),
            pl.BlockSpec((G, H), lambda i: (0, 0)),
            pl.BlockSpec((G, 8), lambda i: (0, 0)),
            pl.BlockSpec((G, H), lambda i: (0, 0)),
            pl.BlockSpec((H,), lambda i: (0,)),
            pl.BlockSpec((H,), lambda i: (0,)),
            pl.BlockSpec((H,), lambda i: (0,)),
            pl.BlockSpec((1, 1, 256), lambda i: (i, 0, 0)),
        ],
        out_specs=pl.BlockSpec((256, H), lambda i: (i, 0)),
        out_shape=jax.ShapeDtypeStruct((NPAD, H), jnp.float32),
    )(y, gsum, cnt, vsum, w, bias, ms, batch3)


def _post2_body(numL_ref, numR_ref, den_ref, b_ref, out_ref):
    den = jnp.sum(den_ref[...], axis=0) + 1e-16
    num = jnp.concatenate([numL_ref[...], numR_ref[...]], axis=1)
    y = num / den[:, None] + b_ref[...][None, :]
    out_ref[...] = jnp.maximum(y, 0.01 * y)


def _post2(numL, numR, den_parts, b):
    return pl.pallas_call(
        _post2_body,
        grid=(NBLK,),
        in_specs=[
            pl.BlockSpec((256, HH), lambda i: (i, 0)),
            pl.BlockSpec((256, HH), lambda i: (i, 0)),
            pl.BlockSpec((32, 256), lambda i: (0, i)),
            pl.BlockSpec((H,), lambda i: (0,)),
        ],
        out_specs=pl.BlockSpec((256, H), lambda i: (i, 0)),
        out_shape=jax.ShapeDtypeStruct((NPAD, H), jnp.float32),
    )(numL, numR, den_parts, b)


def _post3_body(h2_ref, numL_ref, numR_ref, den_ref, b_ref, cnt_ref, bt_ref,
                pool_ref):
    i = pl.program_id(0)
    den = jnp.sum(den_ref[...], axis=0) + 1e-16
    num = jnp.concatenate([numL_ref[...], numR_ref[...]], axis=1)
    y = h2_ref[...] + num / den[:, None] + b_ref[...][None, :]
    oh = _onehot(bt_ref[0, 0, :])
    part = jnp.dot(oh, y, preferred_element_type=jnp.float32)

    @pl.when(i == 0)
    def _():
        pool_ref[...] = jnp.zeros_like(pool_ref)

    pool_ref[...] += part

    @pl.when(i == NBLK - 1)
    def _():
        cnt = jnp.maximum(cnt_ref[...][:, :1], 1.0)
        pool_ref[...] = pool_ref[...] / cnt


def _post3_pool(h2, numL, numR, den_parts, b, cnt, batch3):
    return pl.pallas_call(
        _post3_body,
        grid=(NBLK,),
        in_specs=[
            pl.BlockSpec((256, H), lambda i: (i, 0)),
            pl.BlockSpec((256, HH), lambda i: (i, 0)),
            pl.BlockSpec((256, HH), lambda i: (i, 0)),
            pl.BlockSpec((32, 256), lambda i: (0, i)),
            pl.BlockSpec((H,), lambda i: (0,)),
            pl.BlockSpec((G, 8), lambda i: (0, 0)),
            pl.BlockSpec((1, 1, 256), lambda i: (i, 0, 0)),
        ],
        out_specs=pl.BlockSpec((G, H), lambda i: (0, 0)),
        out_shape=jax.ShapeDtypeStruct((G, H), jnp.float32),
    )(h2, numL, numR, den_parts, b, cnt, batch3)


def _gat_layer(h, W, aS, aD, srcs, dsts, srcs2d, dsts2d):
    hL, hR, sc8 = _matmul_scores(h, W, aS, aD)
    a_src = sc8[:, 0]
    a_dst = sc8[:, 1]
    ex_flat, den_flat = _edge_scalars(a_src, a_dst, srcs, dsts)
    numL, numR = _edge_aggregate(hL, hR, srcs2d, dsts2d, ex_flat)
    return numL, numR, den_flat.reshape(32, NPAD)


def kernel(x, edge_index, batch, W1, aS1, aD1, b1, W2, aS2, aD2, b2,
           W3, aS3, aD3, b3, gn_weight, gn_bias, gn_mean_scale):
    loop = jnp.arange(N, dtype=jnp.int32)
    pad_src = N + (jnp.arange(EPAD - ETOT, dtype=jnp.int32) % (NPAD - N))
    srcs = jnp.concatenate([edge_index[0], loop, pad_src])
    dsts = jnp.concatenate([edge_index[1], loop,
                            jnp.full((EPAD - ETOT,), N, jnp.int32)])
    srcs2d = srcs.reshape(EROWS, C)
    dsts2d = dsts.reshape(EROWS, C)
    batch_pad = jnp.concatenate([batch, jnp.full((NPAD - N,), G, jnp.int32)])
    batch3 = batch_pad.reshape(NBLK, 1, 256)
    x_pad = jnp.pad(x, ((0, NPAD - N), (0, 0)))

    numL, numR, den = _gat_layer(x_pad, W1, aS1, aD1, srcs, dsts, srcs2d, dsts2d)
    y1, gsum, cnt = _post1(numL, numR, den, b1, batch3)
    vsum = _var_pass(y1, gsum, cnt, gn_mean_scale, batch3)
    h1 = _norm_pass(y1, gsum, cnt, vsum, gn_weight, gn_bias,
                    gn_mean_scale, batch3)

    numL, numR, den = _gat_layer(h1, W2, aS2, aD2, srcs, dsts, srcs2d, dsts2d)
    h2 = _post2(numL, numR, den, b2)

    numL, numR, den = _gat_layer(h2, W3, aS3, aD3, srcs, dsts, srcs2d, dsts2d)
    return _post3_pool(h2, numL, numR, den, b3, cnt, batch3)


# speedup vs baseline: 3.9888x; 1.0345x over previous
import functools

import jax
import jax.numpy as jnp
from jax import lax
from jax.experimental import pallas as pl
from jax.experimental.pallas import tpu as pltpu
from jax.experimental.pallas import tpu_sc as plsc

N = 10000
E = 320000
IN = 128
H = 256
G = 64

NPAD = 10240
HH = H // 2
ETOT = E + N
C = 128
EROWS = 2688
EPAD = EROWS * C
EPT_A = EPAD // 32
RB = EROWS // 16
GR = 8
NBLK = NPAD // 256
NPT = NPAD // 16

_SC_PARAMS = pltpu.CompilerParams(needs_layout_passes=False)


def _sc_mesh():
    return plsc.VectorSubcoreMesh(core_axis_name="c", subcore_axis_name="s")


def _edge_scalar_body(as_hbm, ad_hbm, srcsf_hbm, dstsf_hbm, ex_hbm, den_hbm,
                      as_v, ad_v, src_v, dst_v, ex_v, den_v):
    cid = lax.axis_index("c")
    sid = lax.axis_index("s")
    wid = sid * 2 + cid
    pltpu.sync_copy(as_hbm, as_v)
    pltpu.sync_copy(ad_hbm, ad_v)
    pltpu.sync_copy(srcsf_hbm.at[pl.ds(wid * EPT_A, EPT_A)], src_v)
    pltpu.sync_copy(dstsf_hbm.at[pl.ds(wid * EPT_A, EPT_A)], dst_v)

    def zero_body(i, _):
        den_v[pl.ds(i * 16, 16)] = jnp.zeros((16,), jnp.float32)
        return 0
    lax.fori_loop(0, NPAD // 16, zero_body, 0)

    def edge16_body(e, _):
        sl = pl.ds(e * 16, 16)
        src16 = src_v[sl]
        dst16 = dst_v[sl]
        asg = plsc.load_gather(as_v, [src16])
        adg = plsc.load_gather(ad_v, [dst16])
        ee = asg + adg
        ee = jnp.maximum(ee, ee * 0.2)
        ex = jnp.exp(ee)
        ex_v[sl] = ex
        plsc.addupdate_scatter(den_v, [dst16], ex)
        return 0
    lax.fori_loop(0, EPT_A // 16, edge16_body, 0)

    pltpu.sync_copy(ex_v, ex_hbm.at[pl.ds(wid * EPT_A, EPT_A)])
    pltpu.sync_copy(den_v, den_hbm.at[pl.ds(wid * NPAD, NPAD)])


def _edge_scalars(a_src, a_dst, srcs_flat, dsts_flat):
    k = pl.kernel(
        _edge_scalar_body,
        out_type=(
            jax.ShapeDtypeStruct((EPAD,), jnp.float32),
            jax.ShapeDtypeStruct((32 * NPAD,), jnp.float32),
        ),
        mesh=_sc_mesh(),
        compiler_params=_SC_PARAMS,
        scratch_types=[
            pltpu.VMEM((NPAD,), jnp.float32),
            pltpu.VMEM((NPAD,), jnp.float32),
            pltpu.VMEM((EPT_A,), jnp.int32),
            pltpu.VMEM((EPT_A,), jnp.int32),
            pltpu.VMEM((EPT_A,), jnp.float32),
            pltpu.VMEM((NPAD,), jnp.float32),
        ],
    )
    return k(a_src, a_dst, srcs_flat, dsts_flat)


def _edge_agg_body(hL_hbm, hR_hbm, srcs_hbm, dsts_hbm, exf_hbm,
                   numL_hbm, numR_hbm,
                   src_v, dst_v, ex_v, rows0_v, rows1_v, acc_sh,
                   gsem0, gsem1, ssem0, ssem1):
    cid = lax.axis_index("c")
    sid = lax.axis_index("s")
    rows = (rows0_v, rows1_v)
    gsems = (gsem0, gsem1)
    ssems = (ssem0, ssem1)

    def zrow(i, _):
        def zcol(j, _):
            rows0_v[i, pl.ds(j * 16, 16)] = jnp.zeros((16,), jnp.float32)
            return 0
        lax.fori_loop(0, HH // 16, zcol, 0, unroll=True)
        return 0
    lax.fori_loop(0, C, zrow, 0)
    for j in range(NPT // C):
        pltpu.sync_copy(rows0_v, acc_sh.at[pl.ds(sid * NPT + j * C, C)])
    plsc.subcore_barrier()

    def start_gather(c, b):
        @pl.when(cid == 0)
        def _():
            pltpu.async_copy(hL_hbm.at[src_v.at[c]], rows[b], gsems[b])

        @pl.when(cid == 1)
        def _():
            pltpu.async_copy(hR_hbm.at[src_v.at[c]], rows[b], gsems[b])

    def wait_gather(b):
        pltpu.make_async_copy(hL_hbm.at[pl.ds(0, C)], rows[b], gsems[b]).wait()

    def start_scatter(c, b):
        pltpu.async_copy(rows[b], acc_sh.at[dst_v.at[c]], ssems[b], add=True)

    def wait_scatter(b):
        pltpu.make_async_copy(rows[b], acc_sh.at[pl.ds(0, C)], ssems[b]).wait()

    def scale(c, b):
        def edge_body(i, _):
            exv = plsc.load_gather(
                ex_v, [jnp.zeros((16,), jnp.int32) + (c * C + i)])

            def col_body(j, _):
                sl = pl.ds(j * 16, 16)
                rows[b][i, sl] = rows[b][i, sl] * exv
                return 0
            lax.fori_loop(0, HH // 16, col_body, 0, unroll=True)
            return 0
        lax.fori_loop(0, C, edge_body, 0, unroll=4)

    def group_body(g, _):
        row0 = sid * RB + g * GR
        pltpu.sync_copy(srcs_hbm.at[pl.ds(row0, GR)], src_v)
        pltpu.sync_copy(dsts_hbm.at[pl.ds(row0, GR)], dst_v)
        pltpu.sync_copy(exf_hbm.at[pl.ds(row0 * C, GR * C)], ex_v)
        start_gather(0, 0)
        for c in range(GR):
            b = c % 2
            if c + 1 < GR:
                if c >= 1:
                    wait_scatter(1 - b)
                start_gather(c + 1, 1 - b)
            wait_gather(b)
            scale(c, b)
            start_scatter(c, b)
        wait_scatter(0)
        wait_scatter(1)
        return 0
    lax.fori_loop(0, RB // GR, group_body, 0)
    plsc.subcore_barrier()

    @pl.when(cid == 0)
    def _():
        pltpu.sync_copy(acc_sh.at[pl.ds(sid * NPT, NPT)],
                        numL_hbm.at[pl.ds(sid * NPT, NPT)])

    @pl.when(cid == 1)
    def _():
        pltpu.sync_copy(acc_sh.at[pl.ds(sid * NPT, NPT)],
                        numR_hbm.at[pl.ds(sid * NPT, NPT)])


def _edge_aggregate(hL, hR, srcs2d, dsts2d, ex_flat):
    k = pl.kernel(
        _edge_agg_body,
        out_type=(
            jax.ShapeDtypeStruct((NPAD, HH), jnp.float32),
            jax.ShapeDtypeStruct((NPAD, HH), jnp.float32),
        ),
        mesh=_sc_mesh(),
        compiler_params=_SC_PARAMS,
        scratch_types=[
            pltpu.VMEM((GR, C), jnp.int32),
            pltpu.VMEM((GR, C), jnp.int32),
            pltpu.VMEM((GR * C,), jnp.float32),
            pltpu.VMEM((C, HH), jnp.float32),
            pltpu.VMEM((C, HH), jnp.float32),
            pltpu.VMEM_SHARED((NPAD, HH), jnp.float32),
            pltpu.SemaphoreType.DMA,
            pltpu.SemaphoreType.DMA,
            pltpu.SemaphoreType.DMA,
            pltpu.SemaphoreType.DMA,
        ],
    )
    return k(hL, hR, srcs2d, dsts2d, ex_flat)


def _mm_body(x_ref, w_ref, a2_ref, hL_ref, hR_ref, sc_ref):
    h = jnp.dot(x_ref[...], w_ref[...], preferred_element_type=jnp.float32)
    hL_ref[...] = h[:, :HH]
    hR_ref[...] = h[:, HH:]
    sc_ref[...] = jnp.dot(h, a2_ref[...], preferred_element_type=jnp.float32)


def _matmul_scores(x, W, aS, aD):
    K = x.shape[1]
    A2 = jnp.zeros((H, 8), jnp.float32).at[:, 0].set(aS).at[:, 1].set(aD)
    out = pl.pallas_call(
        _mm_body,
        grid=(NBLK,),
        in_specs=[
            pl.BlockSpec((256, K), lambda i: (i, 0)),
            pl.BlockSpec((K, H), lambda i: (0, 0)),
            pl.BlockSpec((H, 8), lambda i: (0, 0)),
        ],
        out_specs=(
            pl.BlockSpec((256, HH), lambda i: (i, 0)),
            pl.BlockSpec((256, HH), lambda i: (i, 0)),
            pl.BlockSpec((256, 8), lambda i: (i, 0)),
        ),
        out_shape=(
            jax.ShapeDtypeStruct((NPAD, HH), jnp.float32),
            jax.ShapeDtypeStruct((NPAD, HH), jnp.float32),
            jax.ShapeDtypeStruct((NPAD, 8), jnp.float32),
        ),
    )(x, W, A2)
    return out


def _onehot(bt):
    return (bt[None, :] == lax.broadcasted_iota(jnp.int32, (G, 256), 0)
            ).astype(jnp.float32)


def _post1_body(numL_ref, numR_ref, den_ref, b_ref, bt_ref,
                y_ref, gsum_ref, cnt_ref):
    i = pl.program_id(0)
    den = jnp.sum(den_ref[...], axis=0) + 1e-16
    num = jnp.concatenate([numL_ref[...], numR_ref[...]], axis=1)
    y = num / den[:, None] + b_ref[...][None, :]
    y = jnp.maximum(y, 0.01 * y)
    y_ref[...] = y
    oh = _onehot(bt_ref[0, 0, :])
    part = jnp.dot(oh, y, preferred_element_type=jnp.float32)
    pcnt = jnp.dot(oh, jnp.ones((256, 8), jnp.float32),
                   preferred_element_type=jnp.float32)

    @pl.when(i == 0)
    def _():
        gsum_ref[...] = jnp.zeros_like(gsum_ref)
        cnt_ref[...] = jnp.zeros_like(cnt_ref)

    gsum_ref[...] += part
    cnt_ref[...] += pcnt


def _post1(numL, numR, den_parts, b, batch3):
    return pl.pallas_call(
        _post1_body,
        grid=(NBLK,),
        in_specs=[
            pl.BlockSpec((256, HH), lambda i: (i, 0)),
            pl.BlockSpec((256, HH), lambda i: (i, 0)),
            pl.BlockSpec((32, 256), lambda i: (0, i)),
            pl.BlockSpec((H,), lambda i: (0,)),
            pl.BlockSpec((1, 1, 256), lambda i: (i, 0, 0)),
        ],
        out_specs=(
            pl.BlockSpec((256, H), lambda i: (i, 0)),
            pl.BlockSpec((G, H), lambda i: (0, 0)),
            pl.BlockSpec((G, 8), lambda i: (0, 0)),
        ),
        out_shape=(
            jax.ShapeDtypeStruct((NPAD, H), jnp.float32),
            jax.ShapeDtypeStruct((G, H), jnp.float32),
            jax.ShapeDtypeStruct((G, 8), jnp.float32),
        ),
    )(numL, numR, den_parts, b, batch3)


def _var_body(y_ref, gsum_ref, cnt_ref, ms_ref, bt_ref, vsum_ref):
    i = pl.program_id(0)
    cnt = jnp.maximum(cnt_ref[...][:, :1], 1.0)
    mean = gsum_ref[...] / cnt
    bt = bt_ref[0, 0, :]
    oh = _onehot(bt)
    mg = jnp.dot(oh.T, mean, preferred_element_type=jnp.float32)
    oc = y_ref[...] - mg * ms_ref[...][None, :]
    part = jnp.dot(oh, oc * oc, preferred_element_type=jnp.float32)

    @pl.when(i == 0)
    def _():
        vsum_ref[...] = jnp.zeros_like(vsum_ref)

    vsum_ref[...] += part


def _var_pass(y, gsum, cnt, ms, batch3):
    return pl.pallas_call(
        _var_body,
        grid=(NBLK,),
        in_specs=[
            pl.BlockSpec((256, H), lambda i: (i, 0)),
            pl.BlockSpec((G, H), lambda i: (0, 0)),
            pl.BlockSpec((G, 8), lambda i: (0, 0)),
            pl.BlockSpec((H,), lambda i: (0,)),
            pl.BlockSpec((1, 1, 256), lambda i: (i, 0, 0)),
        ],
        out_specs=pl.BlockSpec((G, H), lambda i: (0, 0)),
        out_shape=jax.ShapeDtypeStruct((G, H), jnp.float32),
    )(y, gsum, cnt, ms, batch3)


def _norm_body(y_ref, gsum_ref, cnt_ref, vsum_ref, w_ref, bias_ref, ms_ref,
               bt_ref, out_ref):
    cnt = jnp.maximum(cnt_ref[...][:, :1], 1.0)
    mean = gsum_ref[...] / cnt
    std = jnp.sqrt(vsum_ref[...] / cnt + 1e-5)
    bt = bt_ref[0, 0, :]
    oh = _onehot(bt)
    mg = jnp.dot(oh.T, mean, preferred_element_type=jnp.float32)
    sg = jnp.dot(oh.T, std, preferred_element_type=jnp.float32)
    oc = y_ref[...] - mg * ms_ref[...][None, :]
    out = w_ref[...][None, :] * oc / sg + bias_ref[...][None, :]
    out_ref[...] = jnp.where(sg > 0, out, 0.0)


def _norm_pass(y, gsum, cnt, vsum, w, bias, ms, batch3):
    return pl.pallas_call(
        _norm_body,
        grid=(NBLK,),
        in_specs=[
            pl.BlockSpec((256, H), lambda i: (i, 0)),
            pl.BlockSpec((G, H), lambda i: (0, 0)),
            pl.BlockSpec((G, 8), lambda i: (0, 0)),
            pl.BlockSpec((G, H), lambda i: (0, 0)),
            pl.BlockSpec((H,), lambda i: (0,)),
            pl.BlockSpec((H,), lambda i: (0,)),
            pl.BlockSpec((H,), lambda i: (0,)),
            pl.BlockSpec((1, 1, 256), lambda i: (i, 0, 0)),
        ],
        out_specs=pl.BlockSpec((256, H), lambda i: (i, 0)),
        out_shape=jax.ShapeDtypeStruct((NPAD, H), jnp.float32),
    )(y, gsum, cnt, vsum, w, bias, ms, batch3)


def _post2_body(numL_ref, numR_ref, den_ref, b_ref, out_ref):
    den = jnp.sum(den_ref[...], axis=0) + 1e-16
    num = jnp.concatenate([numL_ref[...], numR_ref[...]], axis=1)
    y = num / den[:, None] + b_ref[...][None, :]
    out_ref[...] = jnp.maximum(y, 0.01 * y)


def _post2(numL, numR, den_parts, b):
    return pl.pallas_call(
        _post2_body,
        grid=(NBLK,),
        in_specs=[
            pl.BlockSpec((256, HH), lambda i: (i, 0)),
            pl.BlockSpec((256, HH), lambda i: (i, 0)),
            pl.BlockSpec((32, 256), lambda i: (0, i)),
            pl.BlockSpec((H,), lambda i: (0,)),
        ],
        out_specs=pl.BlockSpec((256, H), lambda i: (i, 0)),
        out_shape=jax.ShapeDtypeStruct((NPAD, H), jnp.float32),
    )(numL, numR, den_parts, b)


def _post3_body(h2_ref, numL_ref, numR_ref, den_ref, b_ref, cnt_ref, bt_ref,
                pool_ref):
    i = pl.program_id(0)
    den = jnp.sum(den_ref[...], axis=0) + 1e-16
    num = jnp.concatenate([numL_ref[...], numR_ref[...]], axis=1)
    y = h2_ref[...] + num / den[:, None] + b_ref[...][None, :]
    oh = _onehot(bt_ref[0, 0, :])
    part = jnp.dot(oh, y, preferred_element_type=jnp.float32)

    @pl.when(i == 0)
    def _():
        pool_ref[...] = jnp.zeros_like(pool_ref)

    pool_ref[...] += part

    @pl.when(i == NBLK - 1)
    def _():
        cnt = jnp.maximum(cnt_ref[...][:, :1], 1.0)
        pool_ref[...] = pool_ref[...] / cnt


def _post3_pool(h2, numL, numR, den_parts, b, cnt, batch3):
    return pl.pallas_call(
        _post3_body,
        grid=(NBLK,),
        in_specs=[
            pl.BlockSpec((256, H), lambda i: (i, 0)),
            pl.BlockSpec((256, HH), lambda i: (i, 0)),
            pl.BlockSpec((256, HH), lambda i: (i, 0)),
            pl.BlockSpec((32, 256), lambda i: (0, i)),
            pl.BlockSpec((H,), lambda i: (0,)),
            pl.BlockSpec((G, 8), lambda i: (0, 0)),
            pl.BlockSpec((1, 1, 256), lambda i: (i, 0, 0)),
        ],
        out_specs=pl.BlockSpec((G, H), lambda i: (0, 0)),
        out_shape=jax.ShapeDtypeStruct((G, H), jnp.float32),
    )(h2, numL, numR, den_parts, b, cnt, batch3)


def _gat_layer(h, W, aS, aD, srcs, dsts, srcs2d, dsts2d):
    hL, hR, sc8 = _matmul_scores(h, W, aS, aD)
    a_src = sc8[:, 0]
    a_dst = sc8[:, 1]
    ex_flat, den_flat = _edge_scalars(a_src, a_dst, srcs, dsts)
    numL, numR = _edge_aggregate(hL, hR, srcs2d, dsts2d, ex_flat)
    return numL, numR, den_flat.reshape(32, NPAD)


def kernel(x, edge_index, batch, W1, aS1, aD1, b1, W2, aS2, aD2, b2,
           W3, aS3, aD3, b3, gn_weight, gn_bias, gn_mean_scale):
    loop = jnp.arange(N, dtype=jnp.int32)
    pad_src = N + (jnp.arange(EPAD - ETOT, dtype=jnp.int32) % (NPAD - N))
    srcs = jnp.concatenate([edge_index[0], loop, pad_src])
    dsts = jnp.concatenate([edge_index[1], loop,
                            jnp.full((EPAD - ETOT,), N, jnp.int32)])
    srcs2d = srcs.reshape(EROWS, C)
    dsts2d = dsts.reshape(EROWS, C)
    batch_pad = jnp.concatenate([batch, jnp.full((NPAD - N,), G, jnp.int32)])
    batch3 = batch_pad.reshape(NBLK, 1, 256)
    x_pad = jnp.pad(x, ((0, NPAD - N), (0, 0)))

    numL, numR, den = _gat_layer(x_pad, W1, aS1, aD1, srcs, dsts, srcs2d, dsts2d)
    y1, gsum, cnt = _post1(numL, numR, den, b1, batch3)
    vsum = _var_pass(y1, gsum, cnt, gn_mean_scale, batch3)
    h1 = _norm_pass(y1, gsum, cnt, vsum, gn_weight, gn_bias,
                    gn_mean_scale, batch3)

    numL, numR, den = _gat_layer(h1, W2, aS2, aD2, srcs, dsts, srcs2d, dsts2d)
    h2 = _post2(numL, numR, den, b2)

    numL, numR, den = _gat_layer(h2, W3, aS3, aD3, srcs, dsts, srcs2d, dsts2d)
    return _post3_pool(h2, numL, numR, den, b3, cnt, batch3)


# speedup vs baseline: 4.2950x; 1.0768x over previous
import functools

import jax
import jax.numpy as jnp
from jax import lax
from jax.experimental import pallas as pl
from jax.experimental.pallas import tpu as pltpu
from jax.experimental.pallas import tpu_sc as plsc

N = 10000
E = 320000
IN = 128
H = 256
G = 64

NPAD = 10240
HH = H // 2
ETOT = E + N
C = 128
EROWS = 2688
EPAD = EROWS * C
EPT_A = EPAD // 32
RB = EROWS // 16
GR = 24
NBLK = NPAD // 256
NPT = NPAD // 16

_SC_PARAMS = pltpu.CompilerParams(needs_layout_passes=False)


def _sc_mesh():
    return plsc.VectorSubcoreMesh(core_axis_name="c", subcore_axis_name="s")


def _edge_scalar_body(as_hbm, ad_hbm, srcsf_hbm, dstsf_hbm, ex_hbm, den_hbm,
                      as_v, ad_v, src_v, dst_v, ex_v, den_v):
    cid = lax.axis_index("c")
    sid = lax.axis_index("s")
    wid = sid * 2 + cid
    pltpu.sync_copy(as_hbm, as_v)
    pltpu.sync_copy(ad_hbm, ad_v)
    pltpu.sync_copy(srcsf_hbm.at[pl.ds(wid * EPT_A, EPT_A)], src_v)
    pltpu.sync_copy(dstsf_hbm.at[pl.ds(wid * EPT_A, EPT_A)], dst_v)

    def zero_body(i, _):
        den_v[pl.ds(i * 16, 16)] = jnp.zeros((16,), jnp.float32)
        return 0
    lax.fori_loop(0, NPAD // 16, zero_body, 0)

    def edge16_body(e, _):
        sl = pl.ds(e * 16, 16)
        src16 = src_v[sl]
        dst16 = dst_v[sl]
        asg = plsc.load_gather(as_v, [src16])
        adg = plsc.load_gather(ad_v, [dst16])
        ee = asg + adg
        ee = jnp.maximum(ee, ee * 0.2)
        ex = jnp.exp(ee)
        ex_v[sl] = ex
        plsc.addupdate_scatter(den_v, [dst16], ex)
        return 0
    lax.fori_loop(0, EPT_A // 16, edge16_body, 0)

    pltpu.sync_copy(ex_v, ex_hbm.at[pl.ds(wid * EPT_A, EPT_A)])
    pltpu.sync_copy(den_v, den_hbm.at[pl.ds(wid * NPAD, NPAD)])


def _edge_scalars(a_src, a_dst, srcs_flat, dsts_flat):
    k = pl.kernel(
        _edge_scalar_body,
        out_type=(
            jax.ShapeDtypeStruct((EPAD,), jnp.float32),
            jax.ShapeDtypeStruct((32 * NPAD,), jnp.float32),
        ),
        mesh=_sc_mesh(),
        compiler_params=_SC_PARAMS,
        scratch_types=[
            pltpu.VMEM((NPAD,), jnp.float32),
            pltpu.VMEM((NPAD,), jnp.float32),
            pltpu.VMEM((EPT_A,), jnp.int32),
            pltpu.VMEM((EPT_A,), jnp.int32),
            pltpu.VMEM((EPT_A,), jnp.float32),
            pltpu.VMEM((NPAD,), jnp.float32),
        ],
    )
    return k(a_src, a_dst, srcs_flat, dsts_flat)


def _edge_agg_body(hL_hbm, hR_hbm, srcs_hbm, dsts_hbm, exf_hbm,
                   numL_hbm, numR_hbm,
                   src_v, dst_v, ex_v, rows0_v, rows1_v, acc_sh,
                   gsem0, gsem1, ssem0, ssem1):
    cid = lax.axis_index("c")
    sid = lax.axis_index("s")
    rows = (rows0_v, rows1_v)
    gsems = (gsem0, gsem1)
    ssems = (ssem0, ssem1)

    def zrow(i, _):
        def zcol(j, _):
            rows0_v[i, pl.ds(j * 16, 16)] = jnp.zeros((16,), jnp.float32)
            return 0
        lax.fori_loop(0, HH // 16, zcol, 0, unroll=True)
        return 0
    lax.fori_loop(0, C, zrow, 0)
    for j in range(NPT // C):
        pltpu.sync_copy(rows0_v, acc_sh.at[pl.ds(sid * NPT + j * C, C)])
    plsc.subcore_barrier()

    def start_gather(c, b):
        @pl.when(cid == 0)
        def _():
            pltpu.async_copy(hL_hbm.at[src_v.at[c]], rows[b], gsems[b])

        @pl.when(cid == 1)
        def _():
            pltpu.async_copy(hR_hbm.at[src_v.at[c]], rows[b], gsems[b])

    def wait_gather(b):
        pltpu.make_async_copy(hL_hbm.at[pl.ds(0, C)], rows[b], gsems[b]).wait()

    def start_scatter(c, b):
        pltpu.async_copy(rows[b], acc_sh.at[dst_v.at[c]], ssems[b], add=True)

    def wait_scatter(b):
        pltpu.make_async_copy(rows[b], acc_sh.at[pl.ds(0, C)], ssems[b]).wait()

    def scale(c, b):
        def edge_body(i, _):
            exv = plsc.load_gather(
                ex_v, [jnp.zeros((16,), jnp.int32) + (c * C + i)])

            def col_body(j, _):
                sl = pl.ds(j * 16, 16)
                rows[b][i, sl] = rows[b][i, sl] * exv
                return 0
            lax.fori_loop(0, HH // 16, col_body, 0, unroll=True)
            return 0
        lax.fori_loop(0, C, edge_body, 0, unroll=4)

    def group_body(g, _):
        row0 = sid * RB + g * GR
        pltpu.sync_copy(srcs_hbm.at[pl.ds(row0, GR)], src_v)
        pltpu.sync_copy(dsts_hbm.at[pl.ds(row0, GR)], dst_v)
        pltpu.sync_copy(exf_hbm.at[pl.ds(row0 * C, GR * C)], ex_v)
        start_gather(0, 0)
        for c in range(GR):
            b = c % 2
            if c + 1 < GR:
                if c >= 1:
                    wait_scatter(1 - b)
                start_gather(c + 1, 1 - b)
            wait_gather(b)
            scale(c, b)
            start_scatter(c, b)
        wait_scatter(0)
        wait_scatter(1)
        return 0
    lax.fori_loop(0, RB // GR, group_body, 0)
    plsc.subcore_barrier()

    @pl.when(cid == 0)
    def _():
        pltpu.sync_copy(acc_sh.at[pl.ds(sid * NPT, NPT)],
                        numL_hbm.at[pl.ds(sid * NPT, NPT)])

    @pl.when(cid == 1)
    def _():
        pltpu.sync_copy(acc_sh.at[pl.ds(sid * NPT, NPT)],
                        numR_hbm.at[pl.ds(sid * NPT, NPT)])


def _edge_aggregate(hL, hR, srcs2d, dsts2d, ex_flat):
    k = pl.kernel(
        _edge_agg_body,
        out_type=(
            jax.ShapeDtypeStruct((NPAD, HH), jnp.float32),
            jax.ShapeDtypeStruct((NPAD, HH), jnp.float32),
        ),
        mesh=_sc_mesh(),
        compiler_params=_SC_PARAMS,
        scratch_types=[
            pltpu.VMEM((GR, C), jnp.int32),
            pltpu.VMEM((GR, C), jnp.int32),
            pltpu.VMEM((GR * C,), jnp.float32),
            pltpu.VMEM((C, HH), jnp.float32),
            pltpu.VMEM((C, HH), jnp.float32),
            pltpu.VMEM_SHARED((NPAD, HH), jnp.float32),
            pltpu.SemaphoreType.DMA,
            pltpu.SemaphoreType.DMA,
            pltpu.SemaphoreType.DMA,
            pltpu.SemaphoreType.DMA,
        ],
    )
    return k(hL, hR, srcs2d, dsts2d, ex_flat)


def _mm_body(x_ref, w_ref, a2_ref, hL_ref, hR_ref, sc_ref):
    h = jnp.dot(x_ref[...], w_ref[...], preferred_element_type=jnp.float32)
    hL_ref[...] = h[:, :HH]
    hR_ref[...] = h[:, HH:]
    sc_ref[...] = jnp.dot(h, a2_ref[...], preferred_element_type=jnp.float32)


def _matmul_scores(x, W, aS, aD):
    K = x.shape[1]
    A2 = jnp.zeros((H, 8), jnp.float32).at[:, 0].set(aS).at[:, 1].set(aD)
    out = pl.pallas_call(
        _mm_body,
        grid=(NBLK,),
        in_specs=[
            pl.BlockSpec((256, K), lambda i: (i, 0)),
            pl.BlockSpec((K, H), lambda i: (0, 0)),
            pl.BlockSpec((H, 8), lambda i: (0, 0)),
        ],
        out_specs=(
            pl.BlockSpec((256, HH), lambda i: (i, 0)),
            pl.BlockSpec((256, HH), lambda i: (i, 0)),
            pl.BlockSpec((256, 8), lambda i: (i, 0)),
        ),
        out_shape=(
            jax.ShapeDtypeStruct((NPAD, HH), jnp.float32),
            jax.ShapeDtypeStruct((NPAD, HH), jnp.float32),
            jax.ShapeDtypeStruct((NPAD, 8), jnp.float32),
        ),
    )(x, W, A2)
    return out


def _onehot(bt):
    return (bt[None, :] == lax.broadcasted_iota(jnp.int32, (G, 256), 0)
            ).astype(jnp.float32)


def _post1_body(numL_ref, numR_ref, den_ref, b_ref, bt_ref,
                y_ref, gsum_ref, cnt_ref):
    i = pl.program_id(0)
    den = jnp.sum(den_ref[...], axis=0) + 1e-16
    num = jnp.concatenate([numL_ref[...], numR_ref[...]], axis=1)
    y = num / den[:, None] + b_ref[...][None, :]
    y = jnp.maximum(y, 0.01 * y)
    y_ref[...] = y
    oh = _onehot(bt_ref[0, 0, :])
    part = jnp.dot(oh, y, preferred_element_type=jnp.float32)
    pcnt = jnp.dot(oh, jnp.ones((256, 8), jnp.float32),
                   preferred_element_type=jnp.float32)

    @pl.when(i == 0)
    def _():
        gsum_ref[...] = jnp.zeros_like(gsum_ref)
        cnt_ref[...] = jnp.zeros_like(cnt_ref)

    gsum_ref[...] += part
    cnt_ref[...] += pcnt


def _post1(numL, numR, den_parts, b, batch3):
    return pl.pallas_call(
        _post1_body,
        grid=(NBLK,),
        in_specs=[
            pl.BlockSpec((256, HH), lambda i: (i, 0)),
            pl.BlockSpec((256, HH), lambda i: (i, 0)),
            pl.BlockSpec((32, 256), lambda i: (0, i)),
            pl.BlockSpec((H,), lambda i: (0,)),
            pl.BlockSpec((1, 1, 256), lambda i: (i, 0, 0)),
        ],
        out_specs=(
            pl.BlockSpec((256, H), lambda i: (i, 0)),
            pl.BlockSpec((G, H), lambda i: (0, 0)),
            pl.BlockSpec((G, 8), lambda i: (0, 0)),
        ),
        out_shape=(
            jax.ShapeDtypeStruct((NPAD, H), jnp.float32),
            jax.ShapeDtypeStruct((G, H), jnp.float32),
            jax.ShapeDtypeStruct((G, 8), jnp.float32),
        ),
    )(numL, numR, den_parts, b, batch3)


def _var_body(y_ref, gsum_ref, cnt_ref, ms_ref, bt_ref, vsum_ref):
    i = pl.program_id(0)
    cnt = jnp.maximum(cnt_ref[...][:, :1], 1.0)
    mean = gsum_ref[...] / cnt
    bt = bt_ref[0, 0, :]
    oh = _onehot(bt)
    mg = jnp.dot(oh.T, mean, preferred_element_type=jnp.float32)
    oc = y_ref[...] - mg * ms_ref[...][None, :]
    part = jnp.dot(oh, oc * oc, preferred_element_type=jnp.float32)

    @pl.when(i == 0)
    def _():
        vsum_ref[...] = jnp.zeros_like(vsum_ref)

    vsum_ref[...] += part


def _var_pass(y, gsum, cnt, ms, batch3):
    return pl.pallas_call(
        _var_body,
        grid=(NBLK,),
        in_specs=[
            pl.BlockSpec((256, H), lambda i: (i, 0)),
            pl.BlockSpec((G, H), lambda i: (0, 0)),
            pl.BlockSpec((G, 8), lambda i: (0, 0)),
            pl.BlockSpec((H,), lambda i: (0,)),
            pl.BlockSpec((1, 1, 256), lambda i: (i, 0, 0)),
        ],
        out_specs=pl.BlockSpec((G, H), lambda i: (0, 0)),
        out_shape=jax.ShapeDtypeStruct((G, H), jnp.float32),
    )(y, gsum, cnt, ms, batch3)


def _norm_body(y_ref, gsum_ref, cnt_ref, vsum_ref, w_ref, bias_ref, ms_ref,
               bt_ref, out_ref):
    cnt = jnp.maximum(cnt_ref[...][:, :1], 1.0)
    mean = gsum_ref[...] / cnt
    std = jnp.sqrt(vsum_ref[...] / cnt + 1e-5)
    bt = bt_ref[0, 0, :]
    oh = _onehot(bt)
    mg = jnp.dot(oh.T, mean, preferred_element_type=jnp.float32)
    sg = jnp.dot(oh.T, std, preferred_element_type=jnp.float32)
    oc = y_ref[...] - mg * ms_ref[...][None, :]
    out = w_ref[...][None, :] * oc / sg + bias_ref[...][None, :]
    out_ref[...] = jnp.where(sg > 0, out, 0.0)


def _norm_pass(y, gsum, cnt, vsum, w, bias, ms, batch3):
    return pl.pallas_call(
        _norm_body,
        grid=(NBLK,),
        in_specs=[
            pl.BlockSpec((256, H), lambda i: (i, 0)),
            pl.BlockSpec((G, H), lambda i: (0, 0)),
            pl.BlockSpec((G, 8), lambda i: (0, 0)),
            pl.BlockSpec((G, H), lambda i: (0, 0)),
            pl.BlockSpec((H,), lambda i: (0,)),
            pl.BlockSpec((H,), lambda i: (0,)),
            pl.BlockSpec((H,), lambda i: (0,)),
            pl.BlockSpec((1, 1, 256), lambda i: (i, 0, 0)),
        ],
        out_specs=pl.BlockSpec((256, H), lambda i: (i, 0)),
        out_shape=jax.ShapeDtypeStruct((NPAD, H), jnp.float32),
    )(y, gsum, cnt, vsum, w, bias, ms, batch3)


def _post2_body(numL_ref, numR_ref, den_ref, b_ref, out_ref):
    den = jnp.sum(den_ref[...], axis=0) + 1e-16
    num = jnp.concatenate([numL_ref[...], numR_ref[...]], axis=1)
    y = num / den[:, None] + b_ref[...][None, :]
    out_ref[...] = jnp.maximum(y, 0.01 * y)


def _post2(numL, numR, den_parts, b):
    return pl.pallas_call(
        _post2_body,
        grid=(NBLK,),
        in_specs=[
            pl.BlockSpec((256, HH), lambda i: (i, 0)),
            pl.BlockSpec((256, HH), lambda i: (i, 0)),
            pl.BlockSpec((32, 256), lambda i: (0, i)),
            pl.BlockSpec((H,), lambda i: (0,)),
        ],
        out_specs=pl.BlockSpec((256, H), lambda i: (i, 0)),
        out_shape=jax.ShapeDtypeStruct((NPAD, H), jnp.float32),
    )(numL, numR, den_parts, b)


def _post3_body(h2_ref, numL_ref, numR_ref, den_ref, b_ref, cnt_ref, bt_ref,
                pool_ref):
    i = pl.program_id(0)
    den = jnp.sum(den_ref[...], axis=0) + 1e-16
    num = jnp.concatenate([numL_ref[...], numR_ref[...]], axis=1)
    y = h2_ref[...] + num / den[:, None] + b_ref[...][None, :]
    oh = _onehot(bt_ref[0, 0, :])
    part = jnp.dot(oh, y, preferred_element_type=jnp.float32)

    @pl.when(i == 0)
    def _():
        pool_ref[...] = jnp.zeros_like(pool_ref)

    pool_ref[...] += part

    @pl.when(i == NBLK - 1)
    def _():
        cnt = jnp.maximum(cnt_ref[...][:, :1], 1.0)
        pool_ref[...] = pool_ref[...] / cnt


def _post3_pool(h2, numL, numR, den_parts, b, cnt, batch3):
    return pl.pallas_call(
        _post3_body,
        grid=(NBLK,),
        in_specs=[
            pl.BlockSpec((256, H), lambda i: (i, 0)),
            pl.BlockSpec((256, HH), lambda i: (i, 0)),
            pl.BlockSpec((256, HH), lambda i: (i, 0)),
            pl.BlockSpec((32, 256), lambda i: (0, i)),
            pl.BlockSpec((H,), lambda i: (0,)),
            pl.BlockSpec((G, 8), lambda i: (0, 0)),
            pl.BlockSpec((1, 1, 256), lambda i: (i, 0, 0)),
        ],
        out_specs=pl.BlockSpec((G, H), lambda i: (0, 0)),
        out_shape=jax.ShapeDtypeStruct((G, H), jnp.float32),
    )(h2, numL, numR, den_parts, b, cnt, batch3)


def _gat_layer(h, W, aS, aD, srcs, dsts, srcs2d, dsts2d):
    hL, hR, sc8 = _matmul_scores(h, W, aS, aD)
    a_src = sc8[:, 0]
    a_dst = sc8[:, 1]
    ex_flat, den_flat = _edge_scalars(a_src, a_dst, srcs, dsts)
    numL, numR = _edge_aggregate(hL, hR, srcs2d, dsts2d, ex_flat)
    return numL, numR, den_flat.reshape(32, NPAD)


def kernel(x, edge_index, batch, W1, aS1, aD1, b1, W2, aS2, aD2, b2,
           W3, aS3, aD3, b3, gn_weight, gn_bias, gn_mean_scale):
    loop = jnp.arange(N, dtype=jnp.int32)
    pad_src = N + (jnp.arange(EPAD - ETOT, dtype=jnp.int32) % (NPAD - N))
    srcs = jnp.concatenate([edge_index[0], loop, pad_src])
    dsts = jnp.concatenate([edge_index[1], loop,
                            jnp.full((EPAD - ETOT,), N, jnp.int32)])
    srcs2d = srcs.reshape(EROWS, C)
    dsts2d = dsts.reshape(EROWS, C)
    batch_pad = jnp.concatenate([batch, jnp.full((NPAD - N,), G, jnp.int32)])
    batch3 = batch_pad.reshape(NBLK, 1, 256)
    x_pad = jnp.pad(x, ((0, NPAD - N), (0, 0)))

    numL, numR, den = _gat_layer(x_pad, W1, aS1, aD1, srcs, dsts, srcs2d, dsts2d)
    y1, gsum, cnt = _post1(numL, numR, den, b1, batch3)
    vsum = _var_pass(y1, gsum, cnt, gn_mean_scale, batch3)
    h1 = _norm_pass(y1, gsum, cnt, vsum, gn_weight, gn_bias,
                    gn_mean_scale, batch3)

    numL, numR, den = _gat_layer(h1, W2, aS2, aD2, srcs, dsts, srcs2d, dsts2d)
    h2 = _post2(numL, numR, den, b2)

    numL, numR, den = _gat_layer(h2, W3, aS3, aD3, srcs, dsts, srcs2d, dsts2d)
    return _post3_pool(h2, numL, numR, den, b3, cnt, batch3)
